# baseline (speedup 1.0000x reference)
import functools

import numpy as np
import jax
import jax.numpy as jnp
from jax import lax
from jax.experimental import pallas as pl
from jax.experimental.pallas import tpu as pltpu
from jax.experimental.pallas import tpu_sc as plsc

_B, _H, _D = 2048, 512, 32
_NW = 32
_SPW = _B // _NW
_NG = _SPW // 16
_LUT = 1024


def _w_table():
    t = np.arange(_LUT, dtype=np.float64)
    t = np.minimum(t, _H)
    w = 1.0 + 0.7 * (t / (_H - 1)) ** 2.5
    return jnp.asarray(w.astype(np.float32))


def _sc_body(pred_hbm, targ_hbm, wlut_hbm, out_hbm,
             wlut_v, tbuf, prow, trow, idx_v, acc_v):
    c = lax.axis_index("c")
    s = lax.axis_index("s")
    wid = s * 2 + c
    base_row = wid * (_SPW * _H)

    pltpu.sync_copy(wlut_hbm, wlut_v)

    lane = lax.iota(jnp.int32, 16)
    ones_i = jnp.full((16,), 1, jnp.int32)
    zeros_i = jnp.full((16,), 0, jnp.int32)
    acc = jnp.zeros((16,), jnp.float32)

    for g in range(_NG):
        rowvec = jnp.zeros((16,), jnp.int32)
        tvec = jnp.zeros((16,), jnp.float32)
        for k in range(16):
            srow = base_row + (g * 16 + k) * _H
            pltpu.sync_copy(targ_hbm.at[pl.ds(srow, _H), pl.ds(0, 16)], tbuf)

            def cbody(ci, cnt):
                vals = plsc.load_gather(tbuf, [ci * 16 + lane, ones_i])
                return cnt + (vals != 0.0).astype(jnp.float32)

            cnt = lax.fori_loop(0, _H // 16, cbody, jnp.zeros((16,), jnp.float32))
            t = jnp.sum(cnt)
            ti = t.astype(jnp.int32)
            safe_row = srow + jnp.maximum(ti - 1, 0)
            sel = lane == k
            rowvec = jnp.where(sel, jnp.full((16,), safe_row, jnp.int32), rowvec)
            tvec = jnp.where(sel, jnp.full((16,), t, jnp.float32), tvec)

        idx_v[...] = rowvec
        pltpu.sync_copy(pred_hbm.at[idx_v], prow)
        pltpu.sync_copy(targ_hbm.at[idx_v], trow)
        p0 = plsc.load_gather(prow, [lane, zeros_i])
        t0 = plsc.load_gather(trow, [lane, zeros_i])
        w = plsc.load_gather(wlut_v, [tvec.astype(jnp.int32)])
        d = p0 - t0
        acc = acc + jnp.where(tvec >= 1.0, w * d * d, jnp.zeros((16,), jnp.float32))

    acc_v[...] = acc
    pltpu.sync_copy(acc_v, out_hbm.at[wid])


def kernel(pred, targ, weights):
    B, H, D = targ.shape
    pred2 = pred.reshape(B * H, D)
    targ2 = targ.reshape(B * H, D)
    wlut = _w_table()

    mesh = plsc.VectorSubcoreMesh(core_axis_name="c", subcore_axis_name="s")
    run = functools.partial(
        pl.kernel,
        mesh=mesh,
        compiler_params=pltpu.CompilerParams(use_tc_tiling_on_sc=False,
                                             needs_layout_passes=False),
        out_type=jax.ShapeDtypeStruct((_NW, 16), jnp.float32),
        scratch_types=[
            pltpu.VMEM((_LUT,), jnp.float32),
            pltpu.VMEM((_H, 16), jnp.float32),
            pltpu.VMEM((16, _D), jnp.float32),
            pltpu.VMEM((16, _D), jnp.float32),
            pltpu.VMEM((16,), jnp.int32),
            pltpu.VMEM((16,), jnp.float32),
        ],
    )(_sc_body)

    partials = run(pred2, targ2, wlut)
    loss = jnp.sum(partials) * (D / B)
    return (loss, {"a0_loss": loss})


# speedup vs baseline: 1.1308x; 1.1308x over previous
import functools

import numpy as np
import jax
import jax.numpy as jnp
from jax import lax
from jax.experimental import pallas as pl
from jax.experimental.pallas import tpu as pltpu
from jax.experimental.pallas import tpu_sc as plsc

_B, _H, _D = 2048, 512, 32
_NW = 32
_SPW = _B // _NW
_HH = _H // 2
_LUT = 1024


def _w_table():
    t = np.arange(_LUT, dtype=np.float64)
    t = np.minimum(t, _H)
    w = 1.0 + 0.7 * (t / (_H - 1)) ** 2.5
    return jnp.asarray(w.astype(np.float32))


def _sc_body(pred_hbm, targ_hbm, wlut_hbm, out_hbm,
             wlut_v, b0, b1, b2, prow, acc_v, s0, s1, s2, psem):
    c = lax.axis_index("c")
    s = lax.axis_index("s")
    wid = s * 2 + c
    base = wid * _SPW

    pltpu.sync_copy(wlut_hbm, wlut_v)

    lane = lax.iota(jnp.int32, 16)
    ones_i = jnp.full((16,), 1, jnp.int32)
    zeros_i = jnp.full((16,), 0, jnp.int32)
    acc = jnp.zeros((16,), jnp.float32)

    bufs = [b0, b1, b2]
    sems = [s0, s1, s2]

    def stage(chunk):
        j, half = divmod(chunk, 2)
        return pltpu.async_copy(
            targ_hbm.at[pl.ds(base + j, 1), pl.ds(half * _HH, _HH), :],
            bufs[chunk % 3], sems[chunk % 3])

    def count(buf):
        def cbody(ci, cnt):
            vals = plsc.load_gather(buf, [zeros_i, ci * 16 + lane, ones_i])
            return cnt + (vals != 0.0).astype(jnp.float32)
        return lax.fori_loop(0, _HH // 16, cbody, jnp.zeros((16,), jnp.float32))

    h0 = stage(0)
    h1 = stage(1)
    handles = {0: h0, 1: h1}

    tvec = jnp.zeros((16,), jnp.float32)
    t0vec = jnp.zeros((16,), jnp.float32)
    poffvec = jnp.zeros((16,), jnp.int32)
    pred_handles = []

    for j in range(_SPW):
        k = j % 16
        if 2 * j + 2 < 2 * _SPW:
            handles[2 * j + 2] = stage(2 * j + 2)

        c0 = bufs[(2 * j) % 3]
        c1 = bufs[(2 * j + 1) % 3]
        handles.pop(2 * j).wait()
        cnt0 = count(c0)
        handles.pop(2 * j + 1).wait()
        cnt1 = count(c1)

        t = jnp.sum(cnt0 + cnt1)
        ti = t.astype(jnp.int32)
        safe = jnp.maximum(ti - 1, 0)
        pbase = (safe // 8) * 8

        lo = jnp.minimum(safe, _HH - 1)
        hi = jnp.maximum(safe - _HH, 0)
        t0a = plsc.load_gather(c0, [zeros_i, jnp.full((16,), lo, jnp.int32),
                                    zeros_i])
        t0b = plsc.load_gather(c1, [zeros_i, jnp.full((16,), hi, jnp.int32),
                                    zeros_i])
        t0 = jnp.where(jnp.full((16,), safe < _HH), t0a, t0b)

        if 2 * j + 3 < 2 * _SPW:
            handles[2 * j + 3] = stage(2 * j + 3)

        sel = lane == k
        t0vec = jnp.where(sel, t0, t0vec)
        tvec = jnp.where(sel, jnp.full((16,), t, jnp.float32), tvec)
        poffvec = jnp.where(sel, jnp.full((16,), safe - pbase, jnp.int32),
                            poffvec)

        pred_handles.append(pltpu.async_copy(
            pred_hbm.at[pl.ds(base + j, 1), pl.ds(pbase, 8), :],
            prow.at[pl.ds(k, 1)], psem))

        if k == 15:
            for h in pred_handles:
                h.wait()
            pred_handles = []
            p0 = plsc.load_gather(prow, [lane, poffvec, zeros_i])
            w = plsc.load_gather(wlut_v, [tvec.astype(jnp.int32)])
            d = p0 - t0vec
            acc = acc + jnp.where(tvec >= 1.0, w * d * d,
                                  jnp.zeros((16,), jnp.float32))
            tvec = jnp.zeros((16,), jnp.float32)
            t0vec = jnp.zeros((16,), jnp.float32)
            poffvec = jnp.zeros((16,), jnp.int32)

    acc_v[pl.ds(0, 16)] = acc
    pltpu.sync_copy(acc_v, out_hbm.at[pl.ds(wid * 128, 128)])


def kernel(pred, targ, weights):
    B, H, D = targ.shape
    wlut = _w_table()

    mesh = plsc.VectorSubcoreMesh(core_axis_name="c", subcore_axis_name="s")
    run = functools.partial(
        pl.kernel,
        mesh=mesh,
        compiler_params=pltpu.CompilerParams(needs_layout_passes=False),
        out_type=jax.ShapeDtypeStruct((_NW * 128,), jnp.float32),
        scratch_types=[
            pltpu.VMEM((_LUT,), jnp.float32),
            pltpu.VMEM((1, _HH, _D), jnp.float32),
            pltpu.VMEM((1, _HH, _D), jnp.float32),
            pltpu.VMEM((1, _HH, _D), jnp.float32),
            pltpu.VMEM((16, 8, _D), jnp.float32),
            pltpu.VMEM((128,), jnp.float32),
            pltpu.SemaphoreType.DMA,
            pltpu.SemaphoreType.DMA,
            pltpu.SemaphoreType.DMA,
            pltpu.SemaphoreType.DMA,
        ],
    )(_sc_body)

    flat = run(pred, targ, wlut)
    partials = flat.reshape(_NW, 128)[:, :16]
    loss = jnp.sum(partials) * (D / B)
    return (loss, {"a0_loss": loss})
